# baseline (speedup 1.0000x reference)
import jax
import jax.numpy as jnp
from jax.experimental import pallas as pl
from jax.experimental.pallas import tpu as pltpu


def _wsad_body(x_ref, we_ref, be_ref, wc1_ref, wc2_ref, wt_ref, bt_ref,
               wcls_ref, bcls_ref, out_ref, acc_feat, acc4):
    t = x_ref.shape[2]
    j = pl.program_id(1)
    nn = pl.num_programs(1)

    @pl.when(j == 0)
    def _init():
        acc_feat[...] = jnp.zeros_like(acc_feat)
        acc4[...] = jnp.zeros_like(acc4)

    xb = x_ref[0, 0]
    h = jnp.dot(xb, we_ref[...], preferred_element_type=jnp.float32)
    h = jnp.maximum(h + be_ref[...], 0.0)

    s = jnp.mean(h, axis=0, keepdims=True)
    c1 = jnp.maximum(
        jnp.dot(s, wc1_ref[...], preferred_element_type=jnp.float32), 0.0)
    catten = jax.nn.sigmoid(
        jnp.dot(c1, wc2_ref[...], preferred_element_type=jnp.float32))

    t_logit = jnp.sum(h * wt_ref[...], axis=1, keepdims=True) + bt_ref[0, 0]
    tatt = jax.nn.sigmoid(t_logit)

    acc_feat[...] += h * catten

    wcls = wcls_ref[...]
    se_logit = jnp.sum(h * (catten * wcls), axis=1, keepdims=True) + bcls_ref[0, 0]
    ss_logit = jnp.sum(h * ((1.0 - catten) * wcls), axis=1, keepdims=True) + bcls_ref[0, 0]
    score_e = jax.nn.sigmoid(se_logit)
    score_s = jax.nn.sigmoid(ss_logit)

    acc4[...] += jnp.concatenate([score_e, score_s, tatt, 1.0 - tatt], axis=1)

    @pl.when(j == nn - 1)
    def _fin():
        k = t // 16 + 1
        inv_n = 1.0 / nn
        a = acc4[...]
        score_e_m = a[:, 0:1] * inv_n
        score_s_m = a[:, 1:2] * inv_n
        te = a[:, 2:3] * inv_n
        ts = a[:, 3:4] * inv_n

        def softmax_col(v):
            e = jnp.exp(v - jnp.max(v))
            return e / jnp.sum(e)

        we_ = softmax_col(te)
        ws_ = softmax_col(ts)
        bag_ee = jnp.sum(score_e_m * we_)
        bag_es = jnp.sum(score_e_m * ws_)
        bag_se = jnp.sum(score_s_m * we_)
        bag_ss = jnp.sum(score_s_m * ws_)

        sc_scaled = score_e_m * te
        fm = acc_feat[...] * inv_n
        mag = jnp.sqrt(jnp.sum(fm * fm, axis=1, keepdims=True))
        rm = mag * sc_scaled

        iota = jax.lax.broadcasted_iota(jnp.int32, (t, 1), 0)
        sels, refs = [], []
        for _ in range(k):
            cur = jnp.max(rm)
            first = jnp.min(jnp.where(rm == cur, iota, t))
            onehot = iota == first
            sels.append(jnp.sum(jnp.where(onehot, sc_scaled, 0.0)))
            refs.append(cur)
            rm = jnp.where(onehot, -jnp.inf, rm)

        row = jnp.concatenate([
            jnp.stack(sels)[None, :],
            jnp.stack(refs)[None, :],
            jnp.stack([bag_ee, bag_es, bag_se, bag_ss])[None, :],
            jnp.zeros((1, 128 - (2 * k + 4)), jnp.float32),
        ], axis=1)
        out_ref[0] = row


@jax.jit
def kernel(x, W_enh, b_enh, Wc1, Wc2, Wt, bt, Wcls, bcls):
    b, n, t, d = x.shape
    dh = W_enh.shape[1]
    dm = Wc1.shape[1]
    k = t // 16 + 1

    out = pl.pallas_call(
        _wsad_body,
        grid=(b, n),
        in_specs=[
            pl.BlockSpec((1, 1, t, d), lambda i, j: (i, j, 0, 0)),
            pl.BlockSpec((d, dh), lambda i, j: (0, 0)),
            pl.BlockSpec((1, dh), lambda i, j: (0, 0)),
            pl.BlockSpec((dh, dm), lambda i, j: (0, 0)),
            pl.BlockSpec((dm, dh), lambda i, j: (0, 0)),
            pl.BlockSpec((1, dh), lambda i, j: (0, 0)),
            pl.BlockSpec((1, 1), lambda i, j: (0, 0)),
            pl.BlockSpec((1, dh), lambda i, j: (0, 0)),
            pl.BlockSpec((1, 1), lambda i, j: (0, 0)),
        ],
        out_specs=pl.BlockSpec((1, 1, 128), lambda i, j: (i, 0, 0)),
        out_shape=jax.ShapeDtypeStruct((b, 1, 128), jnp.float32),
        scratch_shapes=[
            pltpu.VMEM((t, dh), jnp.float32),
            pltpu.VMEM((t, 4), jnp.float32),
        ],
    )(x, W_enh, b_enh.reshape(1, dh), Wc1, Wc2, Wt.reshape(1, dh),
      bt.reshape(1, 1), Wcls.reshape(1, dh), bcls.reshape(1, 1))
    return out[:, 0, :2 * k + 4]
